# baseline (speedup 1.0000x reference)
import functools

import jax
import jax.numpy as jnp
from jax import lax
from jax.experimental import pallas as pl
from jax.experimental.pallas import tpu as pltpu
from jax.experimental.pallas import tpu_sc as plsc

D_MODEL = 64
BATCH = 4096
N_MOTIFS = 200
NUM_WORKERS = 32
BC = 512
CPM = BATCH // BC
N_BLOCKS = N_MOTIFS * CPM
BPW = N_BLOCKS // NUM_WORKERS
LANES = 16
V_PER_BLK = BC // LANES


def _sc_encode(sT, table):
    mesh = plsc.VectorSubcoreMesh(core_axis_name="c", subcore_axis_name="s")

    @functools.partial(
        pl.kernel,
        mesh=mesh,
        out_type=jax.ShapeDtypeStruct((N_MOTIFS, D_MODEL, BATCH), jnp.float32),
        scratch_types=[
            pltpu.VMEM((2, BC), jnp.int32),
            pltpu.VMEM((BC,), jnp.float32),
            pltpu.VMEM((2, D_MODEL, BC), jnp.float32),
            pltpu.VMEM((2, D_MODEL), jnp.float32),
            pltpu.VMEM((D_MODEL,), jnp.float32),
            pltpu.SemaphoreType.DMA,
            pltpu.SemaphoreType.DMA,
        ],
    )
    def k(sT_hbm, table_hbm, out_hbm, s_v, sf_v, buf, tab_v, d_v, sem_s,
          sem_w):
        wid = lax.axis_index("s") * 2 + lax.axis_index("c")
        base = wid * BPW

        pltpu.sync_copy(table_hbm, tab_v)
        for g in range(D_MODEL // LANES):
            sl = pl.ds(g * LANES, LANES)
            d_v[sl] = tab_v[1, sl] - tab_v[0, sl]

        def blk_mc(k_):
            blk = base + k_
            return blk // CPM, (blk % CPM) * BC

        def s_load(k_, b):
            m, c = blk_mc(k_)
            pltpu.async_copy(sT_hbm.at[m, pl.ds(c, BC)], s_v.at[b], sem_s)

        def s_wait(k_, b):
            m, c = blk_mc(k_)
            pltpu.make_async_copy(sT_hbm.at[m, pl.ds(c, BC)], s_v.at[b],
                                  sem_s).wait()

        def writeout(k_, b):
            m, c = blk_mc(k_)
            pltpu.async_copy(buf.at[b], out_hbm.at[m, :, pl.ds(c, BC)], sem_w)

        def writeout_wait(k_, b):
            m, c = blk_mc(k_)
            pltpu.make_async_copy(buf.at[b], out_hbm.at[m, :, pl.ds(c, BC)],
                                  sem_w).wait()

        def compute(b):
            for v in range(V_PER_BLK):
                sl = pl.ds(v * LANES, LANES)
                sf_v[sl] = s_v.at[b][sl].astype(jnp.float32)

            def g_body(g, carry):
                gsl = pl.ds(g * LANES, LANES)
                e0v = tab_v[0, gsl]
                dv = d_v[gsl]
                for j in range(LANES):
                    e0s = jnp.full((LANES,), e0v[j], jnp.float32)
                    des = jnp.full((LANES,), dv[j], jnp.float32)
                    row = buf.at[b].at[g * LANES + j]
                    for v in range(V_PER_BLK):
                        sl = pl.ds(v * LANES, LANES)
                        row[sl] = e0s + des * sf_v[sl]
                return carry

            lax.fori_loop(0, D_MODEL // LANES, g_body, 0)

        s_load(0, 0)

        def body(t, carry):
            for b in (0, 1):
                k_ = 2 * t + b
                nb = 1 - b
                s_wait(k_, b)
                if b == 0:
                    s_load(k_ + 1, nb)
                else:
                    pl.when(t < BPW // 2 - 1)(lambda: s_load(k_ + 1, nb))
                pl.when(t > 0)(lambda: writeout_wait(k_ - 2, b))
                compute(b)
                writeout(k_, b)
            return carry

        lax.fori_loop(0, BPW // 2, body, 0)

        writeout_wait(BPW - 2, 0)
        writeout_wait(BPW - 1, 1)

    return k(sT, table)


def kernel(strands, strand_embed):
    sT = strands.T.astype(jnp.int32)
    out = _sc_encode(sT, strand_embed)
    return jnp.transpose(out, (2, 0, 1))


# speedup vs baseline: 1.6354x; 1.6354x over previous
import functools

import jax
import jax.numpy as jnp
from jax import lax
from jax.experimental import pallas as pl
from jax.experimental.pallas import tpu as pltpu
from jax.experimental.pallas import tpu_sc as plsc

D_MODEL = 64
BATCH = 4096
N_MOTIFS = 200
NUM_WORKERS = 32
BC = 512
CPM = BATCH // BC
N_BLOCKS = N_MOTIFS * CPM
BPW = N_BLOCKS // NUM_WORKERS
LANES = 16
V_PER_BLK = BC // LANES


def _sc_encode(sT, e0b_in, db_in):
    mesh = plsc.VectorSubcoreMesh(core_axis_name="c", subcore_axis_name="s")

    @functools.partial(
        pl.kernel,
        mesh=mesh,
        out_type=jax.ShapeDtypeStruct((N_MOTIFS, D_MODEL, BATCH), jnp.float32),
        scratch_types=[
            pltpu.VMEM((2, BC), jnp.int32),
            pltpu.VMEM((BC,), jnp.float32),
            pltpu.VMEM((2, D_MODEL, BC), jnp.float32),
            pltpu.VMEM((D_MODEL, LANES), jnp.float32),
            pltpu.VMEM((D_MODEL, LANES), jnp.float32),
            pltpu.SemaphoreType.DMA,
            pltpu.SemaphoreType.DMA,
        ],
    )
    def k(sT_hbm, e0b_hbm, db_hbm, out_hbm, s_v, sf_v, buf, e0b, db, sem_s,
          sem_w):
        wid = lax.axis_index("s") * 2 + lax.axis_index("c")
        base = wid * BPW
        full16 = pl.ds(0, LANES)

        pltpu.sync_copy(e0b_hbm, e0b)
        pltpu.sync_copy(db_hbm, db)

        def blk_mc(k_):
            blk = base + k_
            return blk // CPM, (blk % CPM) * BC

        def s_load(k_, b):
            m, c = blk_mc(k_)
            pltpu.async_copy(sT_hbm.at[m, pl.ds(c, BC)], s_v.at[b], sem_s)

        def s_wait(k_, b):
            m, c = blk_mc(k_)
            pltpu.make_async_copy(sT_hbm.at[m, pl.ds(c, BC)], s_v.at[b],
                                  sem_s).wait()

        def writeout(k_, b):
            m, c = blk_mc(k_)
            pltpu.async_copy(buf.at[b], out_hbm.at[m, :, pl.ds(c, BC)], sem_w)

        def writeout_wait(k_, b):
            m, c = blk_mc(k_)
            pltpu.make_async_copy(buf.at[b], out_hbm.at[m, :, pl.ds(c, BC)],
                                  sem_w).wait()

        def compute(b):
            for v in range(V_PER_BLK):
                sl = pl.ds(v * LANES, LANES)
                sf_v[sl] = s_v.at[b][sl].astype(jnp.float32)

            def e_body(e, carry):
                e0s = e0b.at[e][full16]
                des = db.at[e][full16]
                row = buf.at[b].at[e]
                for v in range(V_PER_BLK):
                    sl = pl.ds(v * LANES, LANES)
                    row[sl] = e0s + des * sf_v[sl]
                return carry

            lax.fori_loop(0, D_MODEL, e_body, 0)

        s_load(0, 0)

        def body(t, carry):
            for b in (0, 1):
                k_ = 2 * t + b
                nb = 1 - b
                s_wait(k_, b)
                if b == 0:
                    s_load(k_ + 1, nb)
                else:
                    pl.when(t < BPW // 2 - 1)(lambda: s_load(k_ + 1, nb))
                pl.when(t > 0)(lambda: writeout_wait(k_ - 2, b))
                compute(b)
                writeout(k_, b)
            return carry

        lax.fori_loop(0, BPW // 2, body, 0)

        writeout_wait(BPW - 2, 0)
        writeout_wait(BPW - 1, 1)

    return k(sT, e0b_in, db_in)


def kernel(strands, strand_embed):
    sT = strands.T.astype(jnp.int32)
    e0b = jnp.broadcast_to(strand_embed[0][:, None], (D_MODEL, LANES))
    db = jnp.broadcast_to((strand_embed[1] - strand_embed[0])[:, None],
                          (D_MODEL, LANES))
    out = _sc_encode(sT, e0b, db)
    return jnp.transpose(out, (2, 0, 1))
